# baseline (speedup 1.0000x reference)
import functools

import jax
import jax.numpy as jnp
from jax import lax
from jax.experimental import pallas as pl
from jax.experimental.pallas import tpu as pltpu
from jax.experimental.pallas import tpu_sc as plsc

B = 2
S = 2048
D_MODEL = 4096
NUM_EXPERTS = 8
CAP = 512
C_OUT = CAP - 1
ROW_W = NUM_EXPERTS * C_OUT
T = 512
NT = S // T
NTOK = B * S

NW = 32
TOK_PER_W = NTOK // NW
CHUNK = 8
NCH = TOK_PER_W // CHUNK


def _gate_body(cap_ref, x_ref, w_ref, b_ref, meta_ref, mask_ref, counts_ref):
    i = pl.program_id(1)

    @pl.when(i == 0)
    def _init():
        counts_ref[...] = jnp.zeros_like(counts_ref)

    xb = x_ref[0]
    logits = jnp.dot(xb, w_ref[...], preferred_element_type=jnp.float32)
    logits = logits + b_ref[...]

    m = jnp.max(logits, axis=-1, keepdims=True)
    e = jnp.exp(logits - m)
    p = e / jnp.sum(e, axis=-1, keepdims=True)

    iota_e = lax.broadcasted_iota(jnp.int32, (T, NUM_EXPERTS), 1)
    g0 = jnp.max(p, axis=-1, keepdims=True)
    e0 = jnp.min(jnp.where(p == g0, iota_e, NUM_EXPERTS), axis=-1, keepdims=True)
    oh0 = iota_e == e0
    p1 = jnp.where(oh0, -1.0, p)
    g1 = jnp.max(p1, axis=-1, keepdims=True)
    e1 = jnp.min(jnp.where(p1 == g1, iota_e, NUM_EXPERTS), axis=-1, keepdims=True)
    oh1 = iota_e == e1

    iota_r = lax.broadcasted_iota(jnp.int32, (T, T), 0)
    iota_c = lax.broadcasted_iota(jnp.int32, (T, T), 1)
    tri = (iota_r >= iota_c).astype(jnp.float32)
    c0 = jnp.dot(tri, oh0.astype(jnp.float32), preferred_element_type=jnp.float32)
    c1 = jnp.dot(tri, oh1.astype(jnp.float32), preferred_element_type=jnp.float32)

    carry = counts_ref[...]
    pos0 = c0 + carry[0:1, :]
    pos1 = c1 + carry[1:2, :]
    counts_ref[0:1, :] = pos0[T - 1:T, :]
    counts_ref[1:2, :] = pos1[T - 1:T, :]

    cap = cap_ref[0, 0]
    postok0 = jnp.sum(jnp.where(oh0, pos0, 0.0), axis=-1, keepdims=True).astype(jnp.int32)
    postok1 = jnp.sum(jnp.where(oh1, pos1, 0.0), axis=-1, keepdims=True).astype(jnp.int32)
    valid0 = (postok0 < cap) & (postok0 < CAP) & (g0 != 0.0)
    valid1 = (postok1 < cap) & (postok1 < CAP) & (g1 != 0.0)
    cc0 = jnp.where(valid0, postok0 - 1, -1)
    cc1 = jnp.where(valid1, postok1 - 1, -1)

    gb0 = lax.bitcast_convert_type(g0, jnp.int32)
    gb1 = lax.bitcast_convert_type(g1, jnp.int32)
    pad = jnp.zeros((T, 128 - 6), jnp.int32)
    meta_ref[...] = jnp.concatenate([e0, e1, cc0, cc1, gb0, gb1, pad], axis=1)

    col0 = jnp.where(cc0 < 0, -1, e0 * C_OUT + cc0)
    col1 = jnp.where(cc1 < 0, -1, e1 * C_OUT + cc1)
    iota_col = lax.broadcasted_iota(jnp.int32, (T, ROW_W), 1)
    mask_ref[...] = (iota_col == col0) | (iota_col == col1)


def _sc_body(meta_hbm, out_hbm, meta_v, buf_a, buf_b, sem_a, sem_b):
    wid = lax.axis_index("s") * 2 + lax.axis_index("c")
    base = wid * TOK_PER_W
    bb = base // S
    s_base = base - bb * S

    pltpu.sync_copy(meta_hbm.at[pl.ds(base, TOK_PER_W), :], meta_v)

    zeros16 = jnp.zeros((16,), jnp.float32)

    def _zero(i, c):
        t = lax.shift_right_logical(i, 3)
        ee = lax.bitwise_and(i, 7)
        for l in range(32):
            off = min(l * 16, C_OUT - 16)
            buf_a[t, ee, pl.ds(off, 16)] = zeros16
            buf_b[t, ee, pl.ds(off, 16)] = zeros16
        return c

    lax.fori_loop(0, CHUNK * NUM_EXPERTS, _zero, 0)

    lane16 = lax.iota(jnp.int32, 16)
    tloc = lax.shift_right_logical(lane16, 1)
    kk = lax.bitwise_and(lane16, 1)
    bufs = (buf_a, buf_b)
    sems = (sem_a, sem_b)
    handles = [None, None]

    for c in range(NCH):
        par = c % 2
        buf = bufs[par]
        rows = tloc + c * CHUNK
        if c >= 2:
            handles[par].wait()
            rp = tloc + (c - 2) * CHUNK
            ep = plsc.load_gather(meta_v, [rp, kk])
            cp = plsc.load_gather(meta_v, [rp, kk + 2])
            plsc.store_scatter(buf, [tloc, ep, jnp.maximum(cp, 0)], zeros16,
                               mask=cp >= 0)
        ecc = plsc.load_gather(meta_v, [rows, kk])
        ccc = plsc.load_gather(meta_v, [rows, kk + 2])
        gcc = plsc.bitcast(plsc.load_gather(meta_v, [rows, kk + 4]), jnp.float32)
        plsc.store_scatter(buf, [tloc, ecc, jnp.maximum(ccc, 0)], gcc,
                           mask=ccc >= 0)
        handles[par] = pltpu.async_copy(
            buf, out_hbm.at[bb, pl.ds(s_base + c * CHUNK, CHUNK)], sems[par])
    handles[0].wait()
    handles[1].wait()


_sc_scatter = functools.partial(
    pl.kernel,
    out_type=jax.ShapeDtypeStruct((B, S, NUM_EXPERTS, C_OUT), jnp.float32),
    mesh=plsc.VectorSubcoreMesh(
        core_axis_name="c", subcore_axis_name="s", num_cores=2, num_subcores=16),
    scratch_types=[
        pltpu.VMEM((TOK_PER_W, 128), jnp.int32),
        pltpu.VMEM((CHUNK, NUM_EXPERTS, C_OUT), jnp.float32),
        pltpu.VMEM((CHUNK, NUM_EXPERTS, C_OUT), jnp.float32),
        pltpu.SemaphoreType.DMA,
        pltpu.SemaphoreType.DMA,
    ],
    compiler_params=pltpu.CompilerParams(
        needs_layout_passes=False, use_tc_tiling_on_sc=True),
)(_sc_body)


@jax.jit
def _router(x, gate_weight, gate_bias, expert_capacity):
    cap = jnp.asarray(expert_capacity, jnp.int32).reshape(1, 1)
    bias = gate_bias.reshape(1, NUM_EXPERTS)
    meta, dispatch = pl.pallas_call(
        _gate_body,
        grid=(B, NT),
        in_specs=[
            pl.BlockSpec(memory_space=pltpu.SMEM),
            pl.BlockSpec((1, T, D_MODEL), lambda b, i: (b, i, 0)),
            pl.BlockSpec((D_MODEL, NUM_EXPERTS), lambda b, i: (0, 0)),
            pl.BlockSpec((1, NUM_EXPERTS), lambda b, i: (0, 0)),
        ],
        out_specs=[
            pl.BlockSpec((T, 128), lambda b, i: (b * NT + i, 0)),
            pl.BlockSpec((T, ROW_W), lambda b, i: (b * NT + i, 0)),
        ],
        out_shape=[
            jax.ShapeDtypeStruct((NTOK, 128), jnp.int32),
            jax.ShapeDtypeStruct((NTOK, ROW_W), jnp.bool_),
        ],
        scratch_shapes=[pltpu.VMEM((2, NUM_EXPERTS), jnp.float32)],
        compiler_params=pltpu.CompilerParams(
            dimension_semantics=("arbitrary", "arbitrary"),
        ),
    )(cap, x, gate_weight, bias)

    comb = _sc_scatter(meta)

    return (comb, dispatch.reshape(B, S, NUM_EXPERTS, C_OUT))


def kernel(x, gate_weight, gate_bias, expert_capacity):
    return _router(x, gate_weight, gate_bias, expert_capacity)


# speedup vs baseline: 1.1796x; 1.1796x over previous
import functools

import jax
import jax.numpy as jnp
from jax import lax
from jax.experimental import pallas as pl
from jax.experimental.pallas import tpu as pltpu
from jax.experimental.pallas import tpu_sc as plsc

B = 2
S = 2048
D_MODEL = 4096
NUM_EXPERTS = 8
CAP = 512
C_OUT = CAP - 1
ROW_W = NUM_EXPERTS * C_OUT
T = 512
NT = S // T
NTOK = B * S

NW = 32
TOK_PER_W = NTOK // NW
CHUNK = 8
NCH = TOK_PER_W // CHUNK


def _gate_body(cap_ref, x_ref, w_ref, b_ref, meta_ref, mask_ref, counts_ref):
    i = pl.program_id(1)

    @pl.when(i == 0)
    def _init():
        counts_ref[...] = jnp.zeros_like(counts_ref)

    xb = x_ref[0]
    logits = jnp.dot(xb, w_ref[...], preferred_element_type=jnp.float32)
    logits = logits + b_ref[...]

    m = jnp.max(logits, axis=-1, keepdims=True)
    e = jnp.exp(logits - m)
    p = e / jnp.sum(e, axis=-1, keepdims=True)

    iota_e = lax.broadcasted_iota(jnp.int32, (T, NUM_EXPERTS), 1)
    g0 = jnp.max(p, axis=-1, keepdims=True)
    e0 = jnp.min(jnp.where(p == g0, iota_e, NUM_EXPERTS), axis=-1, keepdims=True)
    oh0 = iota_e == e0
    p1 = jnp.where(oh0, -1.0, p)
    g1 = jnp.max(p1, axis=-1, keepdims=True)
    e1 = jnp.min(jnp.where(p1 == g1, iota_e, NUM_EXPERTS), axis=-1, keepdims=True)
    oh1 = iota_e == e1

    iota_r = lax.broadcasted_iota(jnp.int32, (T, T), 0)
    iota_c = lax.broadcasted_iota(jnp.int32, (T, T), 1)
    tri = (iota_r >= iota_c).astype(jnp.float32)
    c0 = jnp.dot(tri, oh0.astype(jnp.float32), preferred_element_type=jnp.float32)
    c1 = jnp.dot(tri, oh1.astype(jnp.float32), preferred_element_type=jnp.float32)

    carry = counts_ref[...]
    pos0 = c0 + carry[0:1, :]
    pos1 = c1 + carry[1:2, :]
    counts_ref[0:1, :] = pos0[T - 1:T, :]
    counts_ref[1:2, :] = pos1[T - 1:T, :]

    cap = cap_ref[0, 0]
    postok0 = jnp.sum(jnp.where(oh0, pos0, 0.0), axis=-1, keepdims=True).astype(jnp.int32)
    postok1 = jnp.sum(jnp.where(oh1, pos1, 0.0), axis=-1, keepdims=True).astype(jnp.int32)
    valid0 = (postok0 < cap) & (postok0 < CAP) & (g0 != 0.0)
    valid1 = (postok1 < cap) & (postok1 < CAP) & (g1 != 0.0)
    cc0 = jnp.where(valid0, postok0 - 1, -1)
    cc1 = jnp.where(valid1, postok1 - 1, -1)

    gb0 = lax.bitcast_convert_type(g0, jnp.int32)
    gb1 = lax.bitcast_convert_type(g1, jnp.int32)
    pad = jnp.zeros((T, 128 - 6), jnp.int32)
    meta_ref[...] = jnp.concatenate([e0, e1, cc0, cc1, gb0, gb1, pad], axis=1)

    iota_e4 = lax.broadcasted_iota(jnp.int32, (1, T, NUM_EXPERTS, C_OUT), 2)
    iota_c4 = lax.broadcasted_iota(jnp.int32, (1, T, NUM_EXPERTS, C_OUT), 3)
    hit0 = ((iota_e4 == e0.reshape(1, T, 1, 1)) & (iota_c4 == cc0.reshape(1, T, 1, 1))
            & (cc0.reshape(1, T, 1, 1) >= 0))
    hit1 = ((iota_e4 == e1.reshape(1, T, 1, 1)) & (iota_c4 == cc1.reshape(1, T, 1, 1))
            & (cc1.reshape(1, T, 1, 1) >= 0))
    mask_ref[...] = hit0 | hit1


def _sc_body(meta_hbm, out_hbm, meta_v, buf_a, buf_b, sem_a, sem_b):
    wid = lax.axis_index("s") * 2 + lax.axis_index("c")
    base = wid * TOK_PER_W
    bb = base // S
    s_base = base - bb * S

    pltpu.sync_copy(meta_hbm.at[pl.ds(base, TOK_PER_W), :], meta_v)

    zeros16 = jnp.zeros((16,), jnp.float32)

    def _zero(i, c):
        t = lax.shift_right_logical(i, 3)
        ee = lax.bitwise_and(i, 7)
        for l in range(32):
            off = min(l * 16, C_OUT - 16)
            buf_a[t, ee, pl.ds(off, 16)] = zeros16
            buf_b[t, ee, pl.ds(off, 16)] = zeros16
        return c

    lax.fori_loop(0, CHUNK * NUM_EXPERTS, _zero, 0)

    lane16 = lax.iota(jnp.int32, 16)
    tloc = lax.shift_right_logical(lane16, 1)
    kk = lax.bitwise_and(lane16, 1)
    bufs = (buf_a, buf_b)
    sems = (sem_a, sem_b)
    handles = [None, None]

    for c in range(NCH):
        par = c % 2
        buf = bufs[par]
        rows = tloc + c * CHUNK
        if c >= 2:
            handles[par].wait()
            rp = tloc + (c - 2) * CHUNK
            ep = plsc.load_gather(meta_v, [rp, kk])
            cp = plsc.load_gather(meta_v, [rp, kk + 2])
            plsc.store_scatter(buf, [tloc, ep, jnp.maximum(cp, 0)], zeros16,
                               mask=cp >= 0)
        ecc = plsc.load_gather(meta_v, [rows, kk])
        ccc = plsc.load_gather(meta_v, [rows, kk + 2])
        gcc = plsc.bitcast(plsc.load_gather(meta_v, [rows, kk + 4]), jnp.float32)
        plsc.store_scatter(buf, [tloc, ecc, jnp.maximum(ccc, 0)], gcc,
                           mask=ccc >= 0)
        handles[par] = pltpu.async_copy(
            buf, out_hbm.at[bb, pl.ds(s_base + c * CHUNK, CHUNK)], sems[par])
    handles[0].wait()
    handles[1].wait()


_sc_scatter = functools.partial(
    pl.kernel,
    out_type=jax.ShapeDtypeStruct((B, S, NUM_EXPERTS, C_OUT), jnp.float32),
    mesh=plsc.VectorSubcoreMesh(
        core_axis_name="c", subcore_axis_name="s", num_cores=2, num_subcores=16),
    scratch_types=[
        pltpu.VMEM((TOK_PER_W, 128), jnp.int32),
        pltpu.VMEM((CHUNK, NUM_EXPERTS, C_OUT), jnp.float32),
        pltpu.VMEM((CHUNK, NUM_EXPERTS, C_OUT), jnp.float32),
        pltpu.SemaphoreType.DMA,
        pltpu.SemaphoreType.DMA,
    ],
    compiler_params=pltpu.CompilerParams(
        needs_layout_passes=False, use_tc_tiling_on_sc=True),
)(_sc_body)


@jax.jit
def _router(x, gate_weight, gate_bias, expert_capacity):
    cap = jnp.asarray(expert_capacity, jnp.int32).reshape(1, 1)
    bias = gate_bias.reshape(1, NUM_EXPERTS)
    meta, dispatch = pl.pallas_call(
        _gate_body,
        grid=(B, NT),
        in_specs=[
            pl.BlockSpec(memory_space=pltpu.SMEM),
            pl.BlockSpec((1, T, D_MODEL), lambda b, i: (b, i, 0)),
            pl.BlockSpec((D_MODEL, NUM_EXPERTS), lambda b, i: (0, 0)),
            pl.BlockSpec((1, NUM_EXPERTS), lambda b, i: (0, 0)),
        ],
        out_specs=[
            pl.BlockSpec((T, 128), lambda b, i: (b * NT + i, 0)),
            pl.BlockSpec((1, T, NUM_EXPERTS, C_OUT), lambda b, i: (b, i, 0, 0)),
        ],
        out_shape=[
            jax.ShapeDtypeStruct((NTOK, 128), jnp.int32),
            jax.ShapeDtypeStruct((B, S, NUM_EXPERTS, C_OUT), jnp.bool_),
        ],
        scratch_shapes=[pltpu.VMEM((2, NUM_EXPERTS), jnp.float32)],
        compiler_params=pltpu.CompilerParams(
            dimension_semantics=("arbitrary", "arbitrary"),
        ),
    )(cap, x, gate_weight, bias)

    comb = _sc_scatter(meta)

    return (comb, dispatch)


def kernel(x, gate_weight, gate_bias, expert_capacity):
    return _router(x, gate_weight, gate_bias, expert_capacity)
